# Initial kernel scaffold; baseline (speedup 1.0000x reference)
import jax
import jax.numpy as jnp
from jax.experimental import pallas as pl


def kernel(pred, partial):
    raise NotImplementedError("write your pallas kernel here")


# speedup vs baseline: 2.4002x; 2.4002x over previous
import jax
import jax.numpy as jnp
from jax import lax
from jax.experimental import pallas as pl
from jax.experimental.pallas import tpu as pltpu

N = 16384
M = 2048
NT = 512
NUM_TILES = N // NT
NUM_ITER = 4
BASE_ALPHA = 0.1


def _refine_body(pred_t_ref, partial_ref, out_ref, minb, idxb):
    out_ref[0] = pred_t_ref[0]
    px = partial_ref[0, :, 0:1]
    py = partial_ref[0, :, 1:2]
    pz = partial_ref[0, :, 2:3]
    def bf(v):
        return v.astype(jnp.bfloat16).astype(jnp.float32)
    pbx, pby, pbz = bf(px), bf(py), bf(pz)
    b2 = px * px + py * py + pz * pz
    iota = lax.broadcasted_iota(jnp.int32, (M, NT), 0)

    for _ in range(NUM_ITER):
        def pass0(t, acc):
            s = pl.ds(t * NT, NT)
            rx = out_ref[0, 0:1, s]
            ry = out_ref[0, 1:2, s]
            rz = out_ref[0, 2:3, s]
            a2 = rx * rx + ry * ry + rz * rz
            dot = pbx * bf(rx) + pby * bf(ry) + pbz * bf(rz)
            d2 = (a2 + b2) - 2.0 * dot
            d2 = jnp.maximum(d2, 1e-12)
            m = jnp.min(d2, axis=0)
            idx = jnp.min(jnp.where(d2 <= m[None, :], iota, M), axis=0)
            minb[0, s] = m
            idxb[0, s] = idx
            return jnp.maximum(acc, jnp.max(m))

        maxd2 = lax.fori_loop(0, NUM_TILES, pass0, jnp.float32(-jnp.inf))
        maxdist = jnp.sqrt(jnp.maximum(maxd2, 1e-12))
        inv = 1.0 / (maxdist + 1e-6)

        def pass1(t, carry):
            s = pl.ds(t * NT, NT)
            idx = idxb[0, s][None, :]
            eq = iota == idx
            nx = jnp.min(jnp.where(eq, px, jnp.inf), axis=0)
            ny = jnp.min(jnp.where(eq, py, jnp.inf), axis=0)
            nz = jnp.min(jnp.where(eq, pz, jnp.inf), axis=0)
            md = jnp.sqrt(jnp.maximum(minb[0, s], 1e-12))
            alpha = BASE_ALPHA * (2.0 - md * inv)
            rx = out_ref[0, 0, s]
            ry = out_ref[0, 1, s]
            rz = out_ref[0, 2, s]
            out_ref[0, 0, s] = rx + alpha * (nx - rx)
            out_ref[0, 1, s] = ry + alpha * (ny - ry)
            out_ref[0, 2, s] = rz + alpha * (nz - rz)
            return carry

        lax.fori_loop(0, NUM_TILES, pass1, 0)


def _refine(pred_t, partial):
    B = pred_t.shape[0]
    return pl.pallas_call(
        _refine_body,
        grid=(B,),
        in_specs=[
            pl.BlockSpec((1, 3, N), lambda b: (b, 0, 0)),
            pl.BlockSpec((1, M, 3), lambda b: (b, 0, 0)),
        ],
        out_specs=pl.BlockSpec((1, 3, N), lambda b: (b, 0, 0)),
        out_shape=jax.ShapeDtypeStruct((B, 3, N), jnp.float32),
        scratch_shapes=[
            pltpu.VMEM((1, N), jnp.float32),
            pltpu.VMEM((1, N), jnp.int32),
        ],
    )(pred_t, partial)


@jax.jit
def kernel(pred, partial):
    pred_t = jnp.swapaxes(pred, 1, 2)
    out_t = _refine(pred_t, partial)
    return jnp.swapaxes(out_t, 1, 2)


# speedup vs baseline: 3.2641x; 1.3600x over previous
import functools

import jax
import jax.numpy as jnp
from jax import lax
from jax.experimental import pallas as pl
from jax.experimental.pallas import tpu as pltpu
from jax.experimental.pallas import tpu_sc as plsc

N = 16384
M = 2048
NT = 512
NUM_TILES = N // NT
NUM_ITER = 4
BASE_ALPHA = 0.1

NUM_WORKERS = 32
CHUNK = 2 * N // NUM_WORKERS
GRP = 16


def _nn_body(refined_ref, partial_ref, idx_ref, md_ref, mx_ref):
    px = partial_ref[0, :, 0:1]
    py = partial_ref[0, :, 1:2]
    pz = partial_ref[0, :, 2:3]

    def bf(v):
        return v.astype(jnp.bfloat16).astype(jnp.float32)

    pbx, pby, pbz = bf(px), bf(py), bf(pz)
    b2 = px * px + py * py + pz * pz
    iota = lax.broadcasted_iota(jnp.int32, (M, NT), 0)

    def tile(t, acc):
        s = pl.ds(t * NT, NT)
        rx = refined_ref[0, 0:1, s]
        ry = refined_ref[0, 1:2, s]
        rz = refined_ref[0, 2:3, s]
        a2 = rx * rx + ry * ry + rz * rz
        dot = pbx * bf(rx) + pby * bf(ry) + pbz * bf(rz)
        d2 = (a2 + b2) - 2.0 * dot
        d2 = jnp.maximum(d2, 1e-12)
        m = jnp.min(d2, axis=0)
        idx = jnp.min(jnp.where(d2 <= m[None, :], iota, M), axis=0)
        md_ref[0, 0, s] = jnp.sqrt(m)
        idx_ref[0, 0, s] = idx
        return jnp.maximum(acc, jnp.max(m))

    maxd2 = lax.fori_loop(0, NUM_TILES, tile, jnp.float32(-jnp.inf))
    mx_ref[0, 0, :] = jnp.full((128,), jnp.sqrt(maxd2), jnp.float32)


def _nn_search(refined_t, partial):
    B = refined_t.shape[0]
    return pl.pallas_call(
        _nn_body,
        grid=(B,),
        in_specs=[
            pl.BlockSpec((1, 3, N), lambda b: (b, 0, 0)),
            pl.BlockSpec((1, M, 3), lambda b: (b, 0, 0)),
        ],
        out_specs=[
            pl.BlockSpec((1, 1, N), lambda b: (b, 0, 0)),
            pl.BlockSpec((1, 1, N), lambda b: (b, 0, 0)),
            pl.BlockSpec((1, 1, 128), lambda b: (b, 0, 0)),
        ],
        out_shape=[
            jax.ShapeDtypeStruct((B, 1, N), jnp.int32),
            jax.ShapeDtypeStruct((B, 1, N), jnp.float32),
            jax.ShapeDtypeStruct((B, 1, 128), jnp.float32),
        ],
    )(refined_t, partial)


def _sc_update_body(refined_hbm, partial_hbm, idx_hbm, md_hbm, mx_hbm,
                    out_hbm, ptab, rxv, ryv, rzv, idxv, mdv, mxv):
    wid = lax.axis_index("s") * 2 + lax.axis_index("c")
    b = wid // (NUM_WORKERS // 2)
    qbase = wid * CHUNK
    rbase = b * 3 * N + (wid % (NUM_WORKERS // 2)) * CHUNK

    pltpu.sync_copy(partial_hbm.at[pl.ds(b * 3 * M, 3 * M)], ptab)
    pltpu.sync_copy(refined_hbm.at[pl.ds(rbase, CHUNK)], rxv)
    pltpu.sync_copy(refined_hbm.at[pl.ds(rbase + N, CHUNK)], ryv)
    pltpu.sync_copy(refined_hbm.at[pl.ds(rbase + 2 * N, CHUNK)], rzv)
    pltpu.sync_copy(idx_hbm.at[pl.ds(qbase, CHUNK)], idxv)
    pltpu.sync_copy(md_hbm.at[pl.ds(qbase, CHUNK)], mdv)
    pltpu.sync_copy(mx_hbm.at[pl.ds(b * 128, GRP)], mxv)

    denom = mxv[...] + 1e-6

    def step(i, carry):
        s = pl.ds(i * GRP, GRP)
        nn3 = idxv[s] * 3
        nx = plsc.load_gather(ptab, [nn3])
        ny = plsc.load_gather(ptab, [nn3 + 1])
        nz = plsc.load_gather(ptab, [nn3 + 2])
        alpha = BASE_ALPHA * (2.0 - mdv[s] / denom)
        rx, ry, rz = rxv[s], ryv[s], rzv[s]
        rxv[s] = rx + alpha * (nx - rx)
        ryv[s] = ry + alpha * (ny - ry)
        rzv[s] = rz + alpha * (nz - rz)
        return carry

    lax.fori_loop(0, CHUNK // GRP, step, 0)

    pltpu.sync_copy(rxv, out_hbm.at[pl.ds(rbase, CHUNK)])
    pltpu.sync_copy(ryv, out_hbm.at[pl.ds(rbase + N, CHUNK)])
    pltpu.sync_copy(rzv, out_hbm.at[pl.ds(rbase + 2 * N, CHUNK)])


def _sc_update(refined_t, partial, idx, md, mx):
    B = refined_t.shape[0]
    fn = functools.partial(
        pl.kernel,
        out_type=jax.ShapeDtypeStruct((B * 3 * N,), jnp.float32),
        mesh=plsc.VectorSubcoreMesh(core_axis_name="c", subcore_axis_name="s"),
        compiler_params=pltpu.CompilerParams(needs_layout_passes=False),
        scratch_types=[
            pltpu.VMEM((3 * M,), jnp.float32),
            pltpu.VMEM((CHUNK,), jnp.float32),
            pltpu.VMEM((CHUNK,), jnp.float32),
            pltpu.VMEM((CHUNK,), jnp.float32),
            pltpu.VMEM((CHUNK,), jnp.int32),
            pltpu.VMEM((CHUNK,), jnp.float32),
            pltpu.VMEM((GRP,), jnp.float32),
        ],
    )(_sc_update_body)
    out = fn(refined_t.reshape(-1), partial.reshape(-1), idx.reshape(-1),
             md.reshape(-1), mx.reshape(-1))
    return out.reshape(B, 3, N)


@jax.jit
def kernel(pred, partial):
    refined_t = jnp.swapaxes(pred, 1, 2)
    for _ in range(NUM_ITER):
        idx, md, mx = _nn_search(refined_t, partial)
        refined_t = _sc_update(refined_t, partial, idx, md, mx)
    return jnp.swapaxes(refined_t, 1, 2)


# speedup vs baseline: 4.8234x; 1.4777x over previous
import functools

import jax
import jax.numpy as jnp
from jax import lax
from jax.experimental import pallas as pl
from jax.experimental.pallas import tpu as pltpu
from jax.experimental.pallas import tpu_sc as plsc

N = 16384
M = 2048
NT = 512
NUM_TILES = N // NT
NUM_ITER = 4
BASE_ALPHA = 0.1

NUM_WORKERS = 32
CHUNK = 2 * N // NUM_WORKERS
GRP = 16


def _nn_body(refined_ref, partial_ref, idx_ref, md_ref, mx_ref):
    px = partial_ref[0, :, 0:1]
    py = partial_ref[0, :, 1:2]
    pz = partial_ref[0, :, 2:3]

    pb2 = (2.0 * partial_ref[0]).astype(jnp.bfloat16)
    b2 = px * px + py * py + pz * pz
    iota = lax.broadcasted_iota(jnp.int32, (M, NT), 0)

    def tile(t, acc):
        s = pl.ds(t * NT, NT)
        rall = refined_ref[0, :, s]
        rx = rall[0:1]
        ry = rall[1:2]
        rz = rall[2:3]
        a2 = rx * rx + ry * ry + rz * rz
        dot2 = lax.dot_general(pb2, rall.astype(jnp.bfloat16),
                               (((1,), (0,)), ((), ())),
                               preferred_element_type=jnp.float32)
        d2 = (a2 + b2) - dot2
        d2 = jnp.maximum(d2, 1e-12)
        m = jnp.min(d2, axis=0)
        idx = jnp.min(jnp.where(d2 <= m[None, :], iota, M), axis=0)
        md_ref[0, 0, s] = jnp.sqrt(m)
        idx_ref[0, 0, s] = idx
        return jnp.maximum(acc, jnp.max(m))

    maxd2 = lax.fori_loop(0, NUM_TILES, tile, jnp.float32(-jnp.inf))
    mx_ref[0, 0, :] = jnp.full((128,), jnp.sqrt(maxd2), jnp.float32)


def _nn_search(refined_t, partial):
    B = refined_t.shape[0]
    return pl.pallas_call(
        _nn_body,
        grid=(B,),
        in_specs=[
            pl.BlockSpec((1, 3, N), lambda b: (b, 0, 0)),
            pl.BlockSpec((1, M, 3), lambda b: (b, 0, 0)),
        ],
        out_specs=[
            pl.BlockSpec((1, 1, N), lambda b: (b, 0, 0)),
            pl.BlockSpec((1, 1, N), lambda b: (b, 0, 0)),
            pl.BlockSpec((1, 1, 128), lambda b: (b, 0, 0)),
        ],
        out_shape=[
            jax.ShapeDtypeStruct((B, 1, N), jnp.int32),
            jax.ShapeDtypeStruct((B, 1, N), jnp.float32),
            jax.ShapeDtypeStruct((B, 1, 128), jnp.float32),
        ],
    )(refined_t, partial)


def _sc_update_body(refined_hbm, partial_hbm, idx_hbm, md_hbm, mx_hbm,
                    out_hbm, ptab, rxv, ryv, rzv, idxv, mdv, mxv):
    wid = lax.axis_index("s") * 2 + lax.axis_index("c")
    b = wid // (NUM_WORKERS // 2)
    qbase = wid * CHUNK
    rbase = b * 3 * N + (wid % (NUM_WORKERS // 2)) * CHUNK

    pltpu.sync_copy(partial_hbm.at[pl.ds(b * 3 * M, 3 * M)], ptab)
    pltpu.sync_copy(refined_hbm.at[pl.ds(rbase, CHUNK)], rxv)
    pltpu.sync_copy(refined_hbm.at[pl.ds(rbase + N, CHUNK)], ryv)
    pltpu.sync_copy(refined_hbm.at[pl.ds(rbase + 2 * N, CHUNK)], rzv)
    pltpu.sync_copy(idx_hbm.at[pl.ds(qbase, CHUNK)], idxv)
    pltpu.sync_copy(md_hbm.at[pl.ds(qbase, CHUNK)], mdv)
    pltpu.sync_copy(mx_hbm.at[pl.ds(b * 128, GRP)], mxv)

    denom = mxv[...] + 1e-6

    def step(i, carry):
        s = pl.ds(i * GRP, GRP)
        nn3 = idxv[s] * 3
        nx = plsc.load_gather(ptab, [nn3])
        ny = plsc.load_gather(ptab, [nn3 + 1])
        nz = plsc.load_gather(ptab, [nn3 + 2])
        alpha = BASE_ALPHA * (2.0 - mdv[s] / denom)
        rx, ry, rz = rxv[s], ryv[s], rzv[s]
        rxv[s] = rx + alpha * (nx - rx)
        ryv[s] = ry + alpha * (ny - ry)
        rzv[s] = rz + alpha * (nz - rz)
        return carry

    lax.fori_loop(0, CHUNK // GRP, step, 0)

    pltpu.sync_copy(rxv, out_hbm.at[pl.ds(rbase, CHUNK)])
    pltpu.sync_copy(ryv, out_hbm.at[pl.ds(rbase + N, CHUNK)])
    pltpu.sync_copy(rzv, out_hbm.at[pl.ds(rbase + 2 * N, CHUNK)])


def _sc_update(refined_t, partial, idx, md, mx):
    B = refined_t.shape[0]
    fn = functools.partial(
        pl.kernel,
        out_type=jax.ShapeDtypeStruct((B * 3 * N,), jnp.float32),
        mesh=plsc.VectorSubcoreMesh(core_axis_name="c", subcore_axis_name="s"),
        compiler_params=pltpu.CompilerParams(needs_layout_passes=False),
        scratch_types=[
            pltpu.VMEM((3 * M,), jnp.float32),
            pltpu.VMEM((CHUNK,), jnp.float32),
            pltpu.VMEM((CHUNK,), jnp.float32),
            pltpu.VMEM((CHUNK,), jnp.float32),
            pltpu.VMEM((CHUNK,), jnp.int32),
            pltpu.VMEM((CHUNK,), jnp.float32),
            pltpu.VMEM((GRP,), jnp.float32),
        ],
    )(_sc_update_body)
    out = fn(refined_t.reshape(-1), partial.reshape(-1), idx.reshape(-1),
             md.reshape(-1), mx.reshape(-1))
    return out.reshape(B, 3, N)


@jax.jit
def kernel(pred, partial):
    refined_t = jnp.swapaxes(pred, 1, 2)
    for _ in range(NUM_ITER):
        idx, md, mx = _nn_search(refined_t, partial)
        refined_t = _sc_update(refined_t, partial, idx, md, mx)
    return jnp.swapaxes(refined_t, 1, 2)


# speedup vs baseline: 4.9813x; 1.0327x over previous
import functools

import jax
import jax.numpy as jnp
from jax import lax
from jax.experimental import pallas as pl
from jax.experimental.pallas import tpu as pltpu
from jax.experimental.pallas import tpu_sc as plsc

N = 16384
M = 2048
NT = 512
NUM_TILES = N // NT
NUM_ITER = 4
BASE_ALPHA = 0.1

NUM_WORKERS = 32
CHUNK = N // NUM_WORKERS
GRP = 16


def _nn_body(refined_ref, partial_ref, idx_ref, md_ref, mx_ref):
    px = partial_ref[0, :, 0:1]
    py = partial_ref[0, :, 1:2]
    pz = partial_ref[0, :, 2:3]
    pb2 = (2.0 * partial_ref[0]).astype(jnp.bfloat16)
    b2 = px * px + py * py + pz * pz
    iota = lax.broadcasted_iota(jnp.int32, (M, NT), 0)

    def tile(t, acc):
        s = pl.ds(t * NT, NT)
        rall = refined_ref[0, :, s]
        rx = rall[0:1]
        ry = rall[1:2]
        rz = rall[2:3]
        a2 = rx * rx + ry * ry + rz * rz
        dot2 = lax.dot_general(pb2, rall.astype(jnp.bfloat16),
                               (((1,), (0,)), ((), ())),
                               preferred_element_type=jnp.float32)
        d2 = (a2 + b2) - dot2
        m = jnp.min(d2, axis=0)
        idx = jnp.min(jnp.where(d2 <= m[None, :], iota, M), axis=0)
        md_ref[0, 0, s] = jnp.sqrt(jnp.maximum(m, 1e-12))
        idx_ref[0, 0, s] = idx
        return jnp.maximum(acc, jnp.max(m))

    maxd2 = lax.fori_loop(0, NUM_TILES, tile, jnp.float32(-jnp.inf))
    mx_ref[0, 0, :] = jnp.full((128,), jnp.sqrt(jnp.maximum(maxd2, 1e-12)),
                               jnp.float32)


def _nn_search(refined_t, partial):
    return pl.pallas_call(
        _nn_body,
        grid=(1,),
        in_specs=[
            pl.BlockSpec((1, 3, N), lambda b: (b, 0, 0)),
            pl.BlockSpec((1, M, 3), lambda b: (b, 0, 0)),
        ],
        out_specs=[
            pl.BlockSpec((1, 1, N), lambda b: (b, 0, 0)),
            pl.BlockSpec((1, 1, N), lambda b: (b, 0, 0)),
            pl.BlockSpec((1, 1, 128), lambda b: (b, 0, 0)),
        ],
        out_shape=[
            jax.ShapeDtypeStruct((1, 1, N), jnp.int32),
            jax.ShapeDtypeStruct((1, 1, N), jnp.float32),
            jax.ShapeDtypeStruct((1, 1, 128), jnp.float32),
        ],
    )(refined_t, partial)


def _sc_update_body(refined_hbm, partial_hbm, idx_hbm, md_hbm, mx_hbm,
                    out_hbm, ptab, rxv, ryv, rzv, idxv, mdv, mxv):
    wid = lax.axis_index("s") * 2 + lax.axis_index("c")
    qbase = wid * CHUNK

    pltpu.sync_copy(partial_hbm, ptab)
    pltpu.sync_copy(refined_hbm.at[pl.ds(qbase, CHUNK)], rxv)
    pltpu.sync_copy(refined_hbm.at[pl.ds(qbase + N, CHUNK)], ryv)
    pltpu.sync_copy(refined_hbm.at[pl.ds(qbase + 2 * N, CHUNK)], rzv)
    pltpu.sync_copy(idx_hbm.at[pl.ds(qbase, CHUNK)], idxv)
    pltpu.sync_copy(md_hbm.at[pl.ds(qbase, CHUNK)], mdv)
    pltpu.sync_copy(mx_hbm.at[pl.ds(0, GRP)], mxv)

    denom = mxv[...] + 1e-6

    def step(i, carry):
        s = pl.ds(i * GRP, GRP)
        nn3 = idxv[s] * 3
        nx = plsc.load_gather(ptab, [nn3])
        ny = plsc.load_gather(ptab, [nn3 + 1])
        nz = plsc.load_gather(ptab, [nn3 + 2])
        alpha = BASE_ALPHA * (2.0 - mdv[s] / denom)
        rx, ry, rz = rxv[s], ryv[s], rzv[s]
        rxv[s] = rx + alpha * (nx - rx)
        ryv[s] = ry + alpha * (ny - ry)
        rzv[s] = rz + alpha * (nz - rz)
        return carry

    lax.fori_loop(0, CHUNK // GRP, step, 0)

    pltpu.sync_copy(rxv, out_hbm.at[pl.ds(qbase, CHUNK)])
    pltpu.sync_copy(ryv, out_hbm.at[pl.ds(qbase + N, CHUNK)])
    pltpu.sync_copy(rzv, out_hbm.at[pl.ds(qbase + 2 * N, CHUNK)])


def _sc_update(refined_t, partial, idx, md, mx):
    fn = functools.partial(
        pl.kernel,
        out_type=jax.ShapeDtypeStruct((3 * N,), jnp.float32),
        mesh=plsc.VectorSubcoreMesh(core_axis_name="c", subcore_axis_name="s"),
        compiler_params=pltpu.CompilerParams(needs_layout_passes=False),
        scratch_types=[
            pltpu.VMEM((3 * M,), jnp.float32),
            pltpu.VMEM((CHUNK,), jnp.float32),
            pltpu.VMEM((CHUNK,), jnp.float32),
            pltpu.VMEM((CHUNK,), jnp.float32),
            pltpu.VMEM((CHUNK,), jnp.int32),
            pltpu.VMEM((CHUNK,), jnp.float32),
            pltpu.VMEM((GRP,), jnp.float32),
        ],
    )(_sc_update_body)
    out = fn(refined_t.reshape(-1), partial.reshape(-1), idx.reshape(-1),
             md.reshape(-1), mx.reshape(-1))
    return out.reshape(1, 3, N)


@jax.jit
def kernel(pred, partial):
    B = pred.shape[0]
    pred_t = jnp.swapaxes(pred, 1, 2)
    refined = [pred_t[b:b + 1] for b in range(B)]
    parts = [partial[b:b + 1] for b in range(B)]
    for _ in range(NUM_ITER):
        for b in range(B):
            idx, md, mx = _nn_search(refined[b], parts[b])
            refined[b] = _sc_update(refined[b], parts[b], idx, md, mx)
    return jnp.swapaxes(jnp.concatenate(refined, axis=0), 1, 2)
